# baseline (speedup 1.0000x reference)
import functools

import jax
import jax.numpy as jnp
from jax import lax
from jax.experimental import pallas as pl
from jax.experimental.pallas import tpu as pltpu
from jax.experimental.pallas import tpu_sc as plsc

N = 10000
E = 320000
D = 128
GB2 = 256
L = 3
NCLUST = 100
NCPAD = 112

NCORES = 2
NSUB = 16
NW = NCORES * NSUB
DSUB = D // NW
EPW = E // NW
ECHUNK = 20000
NCHUNK = E // ECHUNK
NGROUP = N // 16

def _wid():
    return lax.axis_index("s") * NCORES + lax.axis_index("c")


@functools.cache
def _sc_kernels():
    mesh = plsc.VectorSubcoreMesh(core_axis_name="c", subcore_axis_name="s")
    params = pltpu.CompilerParams(needs_layout_passes=False)

    @functools.partial(
        pl.kernel,
        out_type=(
            jax.ShapeDtypeStruct((NW, N), jnp.float32),
            jax.ShapeDtypeStruct((E,), jnp.int32),
        ),
        mesh=mesh,
        compiler_params=params,
        scratch_types=[
            pltpu.VMEM((1, N), jnp.float32),
            pltpu.VMEM((EPW,), jnp.int32),
            pltpu.VMEM((EPW,), jnp.int32),
            pltpu.VMEM((EPW,), jnp.int32),
        ],
    )
    def edge_prep_kernel(src_hbm, dst_hbm, deg_hbm, epk_hbm,
                         deg_loc, src_buf, dst_buf, pk_buf):
        wid = _wid()
        base = wid * EPW
        pltpu.sync_copy(src_hbm.at[pl.ds(base, EPW)], src_buf)
        pltpu.sync_copy(dst_hbm.at[pl.ds(base, EPW)], dst_buf)
        zero = jnp.zeros((16,), jnp.float32)

        @pl.loop(0, N // 16, unroll=8)
        def _(i):
            deg_loc[0, pl.ds(i * 16, 16)] = zero

        ones = jnp.ones((16,), jnp.float32)
        zvec = jnp.zeros((16,), jnp.int32)

        @plsc.parallel_loop(0, EPW // 16, unroll=8)
        def _(g):
            sl = pl.ds(g * 16, 16)
            sv = src_buf[sl]
            dv = dst_buf[sl]
            pk_buf[sl] = sv | (dv << 16)
            plsc.addupdate_scatter(deg_loc, [zvec, dv], ones)

        pltpu.sync_copy(pk_buf, epk_hbm.at[pl.ds(base, EPW)])
        pltpu.sync_copy(deg_loc, deg_hbm.at[pl.ds(wid, 1)])

    @functools.partial(
        pl.kernel,
        out_type=jax.ShapeDtypeStruct((D, N), jnp.float32),
        mesh=mesh,
        compiler_params=params,
        scratch_types=[
            pltpu.VMEM((DSUB, N), jnp.float32),
            pltpu.VMEM((DSUB, N), jnp.float32),
            pltpu.VMEM((ECHUNK,), jnp.int32),
            pltpu.VMEM((ECHUNK,), jnp.int32),
            pltpu.SemaphoreType.DMA,
            pltpu.SemaphoreType.DMA,
        ],
    )
    def spmm_kernel(x_hbm, epk_hbm, agg_hbm, x_loc, agg_loc,
                    ek_buf0, ek_buf1, sem0, sem1):
        ek_bufs = (ek_buf0, ek_buf1)
        sems = (sem0, sem1)
        wid = _wid()
        row0 = wid * DSUB
        pltpu.sync_copy(x_hbm.at[pl.ds(row0, DSUB)], x_loc)
        zero = jnp.zeros((16,), jnp.float32)

        @plsc.parallel_loop(0, N // 16, unroll=8)
        def _(i):
            sl = pl.ds(i * 16, 16)
            for r in range(DSUB):
                agg_loc[r, sl] = zero

        rvecs = [jnp.full((16,), r, jnp.int32) for r in range(DSUB)]
        mask16 = jnp.full((16,), 0xFFFF, jnp.int32)

        for b in range(2):
            pltpu.async_copy(epk_hbm.at[pl.ds(b * ECHUNK, ECHUNK)],
                             ek_bufs[b], sems[b])

        @pl.loop(0, NCHUNK, step=2)
        def _(k):
            for b in range(2):
                cur = k + b
                pltpu.make_async_copy(
                    epk_hbm.at[pl.ds(0, ECHUNK)], ek_bufs[b], sems[b]
                ).wait()

                @plsc.parallel_loop(0, ECHUNK // 16, unroll=8)
                def _(g):
                    sl = pl.ds(g * 16, 16)
                    pk = ek_bufs[b][sl]
                    sv = pk & mask16
                    dv = lax.shift_right_logical(pk, 16)
                    for r in range(DSUB):
                        v = plsc.load_gather(x_loc, [rvecs[r], sv])
                        plsc.addupdate_scatter(agg_loc, [rvecs[r], dv], v)

                @pl.when(cur + 2 < NCHUNK)
                def _():
                    pltpu.async_copy(
                        epk_hbm.at[pl.ds((cur + 2) * ECHUNK, ECHUNK)],
                        ek_bufs[b], sems[b])

        pltpu.sync_copy(agg_loc, agg_hbm.at[pl.ds(row0, DSUB)])

    @functools.partial(
        pl.kernel,
        out_type=jax.ShapeDtypeStruct((D, N), jnp.float32),
        mesh=mesh,
        compiler_params=params,
        scratch_types=[
            pltpu.VMEM((DSUB, N), jnp.float32),
            pltpu.VMEM((DSUB, N), jnp.float32),
            pltpu.VMEM((N,), jnp.int32),
            pltpu.VMEM((DSUB, NCPAD), jnp.float32),
            pltpu.VMEM((1, NCPAD), jnp.float32),
        ],
    )
    def cluster_kernel(x_hbm, cl_hbm, out_hbm,
                       x_loc, out_loc, cl_buf, sums, cnt):
        wid = _wid()
        row0 = wid * DSUB
        pltpu.sync_copy(x_hbm.at[pl.ds(row0, DSUB)], x_loc)
        pltpu.sync_copy(cl_hbm, cl_buf)
        zero = jnp.zeros((16,), jnp.float32)
        for k in range(NCPAD // 16):
            sl = pl.ds(k * 16, 16)
            cnt[0, sl] = zero
            for r in range(DSUB):
                sums[r, sl] = zero

        ones = jnp.ones((16,), jnp.float32)
        zvec = jnp.zeros((16,), jnp.int32)
        rvecs = [jnp.full((16,), r, jnp.int32) for r in range(DSUB)]

        @plsc.parallel_loop(0, NGROUP, unroll=8)
        def _(g):
            sl = pl.ds(g * 16, 16)
            cv = cl_buf[sl]
            plsc.addupdate_scatter(cnt, [zvec, cv], ones)
            for r in range(DSUB):
                plsc.addupdate_scatter(sums, [rvecs[r], cv], x_loc[r, sl])

        for k in range(NCPAD // 16):
            sl = pl.ds(k * 16, 16)
            inv = 1.0 / jnp.maximum(cnt[0, sl], 1.0)
            for r in range(DSUB):
                sums[r, sl] = sums[r, sl] * inv

        @plsc.parallel_loop(0, NGROUP, unroll=8)
        def _(g):
            sl = pl.ds(g * 16, 16)
            cv = cl_buf[sl]
            for r in range(DSUB):
                out_loc[r, sl] = plsc.load_gather(sums, [rvecs[r], cv])

        pltpu.sync_copy(out_loc, out_hbm.at[pl.ds(row0, DSUB)])

    return edge_prep_kernel, spmm_kernel, cluster_kernel


BC = 2048
NGRID = (N + BC - 1) // BC


def _fc_body(ge_ref, wfc_ref, bfc_ref, degp_ref, gb_ref, inv_ref):
    gb = lax.dot_general(wfc_ref[...], ge_ref[...], (((0,), (1,)), ((), ())),
                         preferred_element_type=jnp.float32)
    gb_ref[...] = gb + bfc_ref[...]
    deg = jnp.sum(degp_ref[...], axis=0, keepdims=True)
    inv_ref[...] = 1.0 / jnp.maximum(deg, 1.0)


def _fc_call(ge, wfc, bfc2, deg_parts):
    return pl.pallas_call(
        _fc_body,
        grid=(NGRID,),
        in_specs=[
            pl.BlockSpec((BC, D), lambda j: (j, 0)),
            pl.BlockSpec((D, GB2), lambda j: (0, 0)),
            pl.BlockSpec((GB2, 1), lambda j: (0, 0)),
            pl.BlockSpec((NW, BC), lambda j: (0, j)),
        ],
        out_specs=[
            pl.BlockSpec((GB2, BC), lambda j: (0, j)),
            pl.BlockSpec((1, BC), lambda j: (0, j)),
        ],
        out_shape=[
            jax.ShapeDtypeStruct((GB2, N), jnp.float32),
            jax.ShapeDtypeStruct((1, N), jnp.float32),
        ],
    )(ge, wfc, bfc2, deg_parts)


def _layer_body(x_ref, agg_ref, gamma_ref, beta_ref, inv_ref, ws_ref, wn_ref,
                bl_ref, out_ref):
    ag = agg_ref[...] * inv_ref[...]
    h = lax.dot_general(ws_ref[...], x_ref[...], (((0,), (0,)), ((), ())),
                        preferred_element_type=jnp.float32)
    h = h + lax.dot_general(wn_ref[...], ag, (((0,), (0,)), ((), ())),
                            preferred_element_type=jnp.float32)
    h = h + bl_ref[...]
    h = gamma_ref[...] * h + beta_ref[...]
    out_ref[...] = jnp.maximum(h, 0.0)


def _layer_call(X, agg, gbT, inv, ws, wn, bl2):
    return pl.pallas_call(
        _layer_body,
        grid=(NGRID,),
        in_specs=[
            pl.BlockSpec((D, BC), lambda j: (0, j)),
            pl.BlockSpec((D, BC), lambda j: (0, j)),
            pl.BlockSpec((D, BC), lambda j: (0, j)),
            pl.BlockSpec((D, BC), lambda j: (1, j)),
            pl.BlockSpec((1, BC), lambda j: (0, j)),
            pl.BlockSpec((D, D), lambda j: (0, 0)),
            pl.BlockSpec((D, D), lambda j: (0, 0)),
            pl.BlockSpec((D, 1), lambda j: (0, 0)),
        ],
        out_specs=pl.BlockSpec((D, BC), lambda j: (0, j)),
        out_shape=jax.ShapeDtypeStruct((D, N), jnp.float32),
    )(X, agg, gbT, gbT, inv, ws, wn, bl2)


def kernel(x, edge_index, global_embedding, cluster_assignment, W_fc, b_fc,
           Ws, Wn, bl):
    edge_prep_kernel, spmm_kernel, cluster_kernel = _sc_kernels()
    src = edge_index[0]
    dst = edge_index[1]
    X = x.T
    deg_parts, epk = edge_prep_kernel(src, dst)
    gbT, inv = _fc_call(global_embedding, W_fc, b_fc.reshape(GB2, 1), deg_parts)
    for l in range(L):
        agg = spmm_kernel(X, epk)
        X = _layer_call(X, agg, gbT, inv, Ws[l], Wn[l], bl[l].reshape(D, 1))
    outT = cluster_kernel(X, cluster_assignment)
    return outT.T


# speedup vs baseline: 1.0592x; 1.0592x over previous
import functools

import jax
import jax.numpy as jnp
from jax import lax
from jax.experimental import pallas as pl
from jax.experimental.pallas import tpu as pltpu
from jax.experimental.pallas import tpu_sc as plsc

N = 10000
E = 320000
D = 128
GB2 = 256
L = 3
NCLUST = 100
NCPAD = 112

NCORES = 2
NSUB = 16
NW = NCORES * NSUB
DSUB = D // NW
EPW = E // NW
ECHUNK = 16000
NCHUNK = E // ECHUNK
NGROUP = N // 16

def _wid():
    return lax.axis_index("s") * NCORES + lax.axis_index("c")


@functools.cache
def _sc_kernels():
    mesh = plsc.VectorSubcoreMesh(core_axis_name="c", subcore_axis_name="s")
    params = pltpu.CompilerParams(needs_layout_passes=False)

    @functools.partial(
        pl.kernel,
        out_type=(
            jax.ShapeDtypeStruct((NW, N), jnp.float32),
            jax.ShapeDtypeStruct((E,), jnp.int32),
        ),
        mesh=mesh,
        compiler_params=params,
        scratch_types=[
            pltpu.VMEM((1, N), jnp.float32),
            pltpu.VMEM((EPW,), jnp.int32),
            pltpu.VMEM((EPW,), jnp.int32),
            pltpu.VMEM((EPW,), jnp.int32),
        ],
    )
    def edge_prep_kernel(src_hbm, dst_hbm, deg_hbm, epk_hbm,
                         deg_loc, src_buf, dst_buf, pk_buf):
        wid = _wid()
        base = wid * EPW
        pltpu.sync_copy(src_hbm.at[pl.ds(base, EPW)], src_buf)
        pltpu.sync_copy(dst_hbm.at[pl.ds(base, EPW)], dst_buf)
        zero = jnp.zeros((16,), jnp.float32)

        @pl.loop(0, N // 16, unroll=8)
        def _(i):
            deg_loc[0, pl.ds(i * 16, 16)] = zero

        ones = jnp.ones((16,), jnp.float32)
        zvec = jnp.zeros((16,), jnp.int32)

        @plsc.parallel_loop(0, EPW // 16, unroll=8)
        def _(g):
            sl = pl.ds(g * 16, 16)
            sv = src_buf[sl]
            dv = dst_buf[sl]
            pk_buf[sl] = sv | (dv << 16)
            plsc.addupdate_scatter(deg_loc, [zvec, dv], ones)

        pltpu.sync_copy(pk_buf, epk_hbm.at[pl.ds(base, EPW)])
        pltpu.sync_copy(deg_loc, deg_hbm.at[pl.ds(wid, 1)])

    @functools.partial(
        pl.kernel,
        out_type=jax.ShapeDtypeStruct((D, N), jnp.float32),
        mesh=mesh,
        compiler_params=params,
        scratch_types=[
            pltpu.VMEM((DSUB, N), jnp.float32),
            pltpu.VMEM((DSUB, N), jnp.float32),
            pltpu.VMEM((ECHUNK,), jnp.int32),
            pltpu.VMEM((ECHUNK,), jnp.int32),
            pltpu.SemaphoreType.DMA,
            pltpu.SemaphoreType.DMA,
        ],
    )
    def spmm_kernel(x_hbm, epk_hbm, agg_hbm, x_loc, agg_loc,
                    ek_buf0, ek_buf1, sem0, sem1):
        ek_bufs = (ek_buf0, ek_buf1)
        sems = (sem0, sem1)
        wid = _wid()
        row0 = wid * DSUB
        pltpu.sync_copy(x_hbm.at[pl.ds(row0, DSUB)], x_loc)
        zero = jnp.zeros((16,), jnp.float32)

        @plsc.parallel_loop(0, N // 16, unroll=8)
        def _(i):
            sl = pl.ds(i * 16, 16)
            for r in range(DSUB):
                agg_loc[r, sl] = zero

        rvecs = [jnp.full((16,), r, jnp.int32) for r in range(DSUB)]
        mask16 = jnp.full((16,), 0xFFFF, jnp.int32)

        for b in range(2):
            pltpu.async_copy(epk_hbm.at[pl.ds(b * ECHUNK, ECHUNK)],
                             ek_bufs[b], sems[b])

        @pl.loop(0, NCHUNK, step=2)
        def _(k):
            for b in range(2):
                cur = k + b
                pltpu.make_async_copy(
                    epk_hbm.at[pl.ds(0, ECHUNK)], ek_bufs[b], sems[b]
                ).wait()

                @plsc.parallel_loop(0, ECHUNK // 16, unroll=8)
                def _(g):
                    sl = pl.ds(g * 16, 16)
                    pk = ek_bufs[b][sl]
                    sv = pk & mask16
                    dv = lax.shift_right_logical(pk, 16)
                    for r in range(DSUB):
                        v = plsc.load_gather(x_loc, [rvecs[r], sv])
                        plsc.addupdate_scatter(agg_loc, [rvecs[r], dv], v)

                @pl.when(cur + 2 < NCHUNK)
                def _():
                    pltpu.async_copy(
                        epk_hbm.at[pl.ds((cur + 2) * ECHUNK, ECHUNK)],
                        ek_bufs[b], sems[b])

        pltpu.sync_copy(agg_loc, agg_hbm.at[pl.ds(row0, DSUB)])

    return edge_prep_kernel, spmm_kernel


BC = 2048
NGRID = (N + BC - 1) // BC


def _fc_body(ge_ref, wfc_ref, bfc_ref, degp_ref, gb_ref, inv_ref):
    gb = lax.dot_general(wfc_ref[...], ge_ref[...], (((0,), (1,)), ((), ())),
                         preferred_element_type=jnp.float32)
    gb_ref[...] = gb + bfc_ref[...]
    deg = jnp.sum(degp_ref[...], axis=0, keepdims=True)
    inv_ref[...] = 1.0 / jnp.maximum(deg, 1.0)


def _fc_call(ge, wfc, bfc2, deg_parts):
    return pl.pallas_call(
        _fc_body,
        grid=(NGRID,),
        in_specs=[
            pl.BlockSpec((BC, D), lambda j: (j, 0)),
            pl.BlockSpec((D, GB2), lambda j: (0, 0)),
            pl.BlockSpec((GB2, 1), lambda j: (0, 0)),
            pl.BlockSpec((NW, BC), lambda j: (0, j)),
        ],
        out_specs=[
            pl.BlockSpec((GB2, BC), lambda j: (0, j)),
            pl.BlockSpec((1, BC), lambda j: (0, j)),
        ],
        out_shape=[
            jax.ShapeDtypeStruct((GB2, N), jnp.float32),
            jax.ShapeDtypeStruct((1, N), jnp.float32),
        ],
    )(ge, wfc, bfc2, deg_parts)


def _layer_body(x_ref, agg_ref, gamma_ref, beta_ref, inv_ref, ws_ref, wn_ref,
                bl_ref, out_ref):
    ag = agg_ref[...] * inv_ref[...]
    h = lax.dot_general(ws_ref[...], x_ref[...], (((0,), (0,)), ((), ())),
                        preferred_element_type=jnp.float32)
    h = h + lax.dot_general(wn_ref[...], ag, (((0,), (0,)), ((), ())),
                            preferred_element_type=jnp.float32)
    h = h + bl_ref[...]
    h = gamma_ref[...] * h + beta_ref[...]
    out_ref[...] = jnp.maximum(h, 0.0)


def _layer_call(X, agg, gbT, inv, ws, wn, bl2):
    return pl.pallas_call(
        _layer_body,
        grid=(NGRID,),
        in_specs=[
            pl.BlockSpec((D, BC), lambda j: (0, j)),
            pl.BlockSpec((D, BC), lambda j: (0, j)),
            pl.BlockSpec((D, BC), lambda j: (0, j)),
            pl.BlockSpec((D, BC), lambda j: (1, j)),
            pl.BlockSpec((1, BC), lambda j: (0, j)),
            pl.BlockSpec((D, D), lambda j: (0, 0)),
            pl.BlockSpec((D, D), lambda j: (0, 0)),
            pl.BlockSpec((D, 1), lambda j: (0, 0)),
        ],
        out_specs=pl.BlockSpec((D, BC), lambda j: (0, j)),
        out_shape=jax.ShapeDtypeStruct((D, N), jnp.float32),
    )(X, agg, gbT, gbT, inv, ws, wn, bl2)


def _last_layer_body(x_ref, agg_ref, gamma_ref, beta_ref, inv_ref, ws_ref,
                     wn_ref, bl_ref, cl_ref, csum_ref, cnt_ref):
    j = pl.program_id(0)
    ag = agg_ref[...] * inv_ref[...]
    h = lax.dot_general(ws_ref[...], x_ref[...], (((0,), (0,)), ((), ())),
                        preferred_element_type=jnp.float32)
    h = h + lax.dot_general(wn_ref[...], ag, (((0,), (0,)), ((), ())),
                            preferred_element_type=jnp.float32)
    h = h + bl_ref[...]
    h = gamma_ref[...] * h + beta_ref[...]
    h = jnp.maximum(h, 0.0)
    cl = cl_ref[...][0][:, None]
    kio = lax.broadcasted_iota(jnp.int32, (BC, D), 1)
    nio = lax.broadcasted_iota(jnp.int32, (BC, D), 0) + j * BC
    oh = jnp.where((cl == kio) & (nio < N), 1.0, 0.0)
    ps = lax.dot_general(h, oh, (((1,), (0,)), ((), ())),
                         preferred_element_type=jnp.float32)
    pc = jnp.sum(oh, axis=0, keepdims=True)

    @pl.when(j == 0)
    def _():
        csum_ref[...] = jnp.zeros_like(csum_ref)
        cnt_ref[...] = jnp.zeros_like(cnt_ref)

    csum_ref[...] += ps
    cnt_ref[...] += pc


def _last_layer_call(X, agg, gbT, inv, ws, wn, bl2, cl2):
    return pl.pallas_call(
        _last_layer_body,
        grid=(NGRID,),
        in_specs=[
            pl.BlockSpec((D, BC), lambda j: (0, j)),
            pl.BlockSpec((D, BC), lambda j: (0, j)),
            pl.BlockSpec((D, BC), lambda j: (0, j)),
            pl.BlockSpec((D, BC), lambda j: (1, j)),
            pl.BlockSpec((1, BC), lambda j: (0, j)),
            pl.BlockSpec((D, D), lambda j: (0, 0)),
            pl.BlockSpec((D, D), lambda j: (0, 0)),
            pl.BlockSpec((D, 1), lambda j: (0, 0)),
            pl.BlockSpec((1, BC), lambda j: (0, j)),
        ],
        out_specs=[
            pl.BlockSpec((D, D), lambda j: (0, 0)),
            pl.BlockSpec((1, D), lambda j: (0, 0)),
        ],
        out_shape=[
            jax.ShapeDtypeStruct((D, D), jnp.float32),
            jax.ShapeDtypeStruct((1, D), jnp.float32),
        ],
    )(X, agg, gbT, gbT, inv, ws, wn, bl2, cl2)


def _unpool_body(cl_ref, csum_ref, cnt_ref, out_ref):
    m = csum_ref[...] / jnp.maximum(cnt_ref[...], 1.0)
    cl = cl_ref[...][0][:, None]
    kio = lax.broadcasted_iota(jnp.int32, (BC, D), 1)
    oh = jnp.where(cl == kio, 1.0, 0.0)
    out_ref[...] = lax.dot_general(oh, m, (((1,), (1,)), ((), ())),
                                   preferred_element_type=jnp.float32)


def _unpool_call(cl2, csum, cnt):
    return pl.pallas_call(
        _unpool_body,
        grid=(NGRID,),
        in_specs=[
            pl.BlockSpec((1, BC), lambda j: (0, j)),
            pl.BlockSpec((D, D), lambda j: (0, 0)),
            pl.BlockSpec((1, D), lambda j: (0, 0)),
        ],
        out_specs=pl.BlockSpec((BC, D), lambda j: (j, 0)),
        out_shape=jax.ShapeDtypeStruct((N, D), jnp.float32),
    )(cl2, csum, cnt)


def kernel(x, edge_index, global_embedding, cluster_assignment, W_fc, b_fc,
           Ws, Wn, bl):
    edge_prep_kernel, spmm_kernel = _sc_kernels()
    src = edge_index[0]
    dst = edge_index[1]
    cl2 = cluster_assignment.reshape(1, N)
    X = x.T
    deg_parts, epk = edge_prep_kernel(src, dst)
    gbT, inv = _fc_call(global_embedding, W_fc, b_fc.reshape(GB2, 1), deg_parts)
    for l in range(L - 1):
        agg = spmm_kernel(X, epk)
        X = _layer_call(X, agg, gbT, inv, Ws[l], Wn[l], bl[l].reshape(D, 1))
    agg = spmm_kernel(X, epk)
    csum, cnt = _last_layer_call(X, agg, gbT, inv, Ws[L - 1], Wn[L - 1],
                                 bl[L - 1].reshape(D, 1), cl2)
    return _unpool_call(cl2, csum, cnt)
